# Initial kernel scaffold; baseline (speedup 1.0000x reference)
import jax
import jax.numpy as jnp
from jax.experimental import pallas as pl


def kernel(predicted_locations, predicted_scores, boxes, labels, priors_cxcy):
    raise NotImplementedError("write your pallas kernel here")


# speedup vs baseline: 3.4270x; 3.4270x over previous
import jax
import jax.numpy as jnp
from jax import lax
from jax.experimental import pallas as pl
from jax.experimental.pallas import tpu as pltpu

_THRESHOLD = 0.5
_NEG_POS_RATIO = 3.0
_ALPHA = 1.0


def _mbl_kernel(plt_ref, ps_ref, boxes_ref, labels_ref, pxy_ref, pcxcy_ref,
                out_ref):
    f32 = jnp.float32
    bx = boxes_ref[0]
    nobj = bx.shape[0]
    n_p = pxy_ref.shape[1]
    b1 = bx[:, 0:1]
    b2 = bx[:, 1:2]
    b3 = bx[:, 2:3]
    b4 = bx[:, 3:4]
    px1 = pxy_ref[0:1, :]
    py1 = pxy_ref[1:2, :]
    px2 = pxy_ref[2:3, :]
    py2 = pxy_ref[3:4, :]

    ix = jnp.maximum(jnp.minimum(b3, px2) - jnp.maximum(b1, px1), 0.0)
    iy = jnp.maximum(jnp.minimum(b4, py2) - jnp.maximum(b2, py1), 0.0)
    inter = ix * iy
    a1 = (b3 - b1) * (b4 - b2)
    a2 = (px2 - px1) * (py2 - py1)
    ov = inter / (a1 + a2 - inter)

    o_iota = lax.broadcasted_iota(jnp.int32, (nobj, 1), 0).astype(f32)
    p_iota = lax.broadcasted_iota(jnp.int32, (1, n_p), 1).astype(f32)

    ov_max0 = jnp.max(ov, axis=0, keepdims=True)
    ofe = jnp.min(jnp.where(ov == ov_max0, o_iota, float(nobj)),
                  axis=0, keepdims=True)
    ov_max1 = jnp.max(ov, axis=1, keepdims=True)
    pfeo = jnp.min(jnp.where(ov == ov_max1, p_iota, float(n_p) + 1.0),
                   axis=1, keepdims=True)

    eq_sc = p_iota == pfeo
    new_o = jnp.max(jnp.where(eq_sc, o_iota, -1.0), axis=0, keepdims=True)
    has = new_o >= 0.0
    ofe = jnp.where(has, new_o, ofe)
    ov_fe = jnp.where(has, 1.0, ov_max0)

    eq2 = ofe == o_iota

    def gather(v):
        return jnp.sum(jnp.where(eq2, v, 0.0), axis=0, keepdims=True)

    lab = gather(labels_ref[0])
    gcx = gather((b1 + b3) * 0.5)
    gcy = gather((b2 + b4) * 0.5)
    gw = gather(b3 - b1)
    gh = gather(b4 - b2)

    lab = jnp.where(ov_fe < _THRESHOLD, 0.0, lab)
    pos = jnp.where(lab > 0.5, 1.0, 0.0)
    n_pos = jnp.sum(pos)

    pcx = pcxcy_ref[0:1, :]
    pcy = pcxcy_ref[1:2, :]
    pw = pcxcy_ref[2:3, :]
    ph = pcxcy_ref[3:4, :]
    g1 = (gcx - pcx) / (pw * 0.1)
    g2 = (gcy - pcy) / (ph * 0.1)
    g3 = jnp.log(gw / pw) * 5.0
    g4 = jnp.log(gh / ph) * 5.0

    pl_t = plt_ref[0]
    loc = (jnp.abs(pl_t[0:1, :] - g1) + jnp.abs(pl_t[1:2, :] - g2) +
           jnp.abs(pl_t[2:3, :] - g3) + jnp.abs(pl_t[3:4, :] - g4))
    loc_sum = jnp.sum(loc * pos)

    lab_col = lab.reshape(n_p, 1)
    pos_col = pos.reshape(n_p, 1)
    ps = ps_ref[0]
    m = jnp.max(ps, axis=1, keepdims=True)
    logz = m + jnp.log(jnp.sum(jnp.exp(ps - m), axis=1, keepdims=True))
    c_iota = lax.broadcasted_iota(jnp.int32, (1, ps.shape[1]), 1).astype(f32)
    gt_logit = jnp.sum(jnp.where(lab_col == c_iota, ps, 0.0),
                       axis=1, keepdims=True)
    conf = logz - gt_logit
    conf_pos_sum = jnp.sum(conf * pos_col)
    conf_neg = jnp.where(pos_col > 0.0, 0.0, conf)

    k = jnp.minimum(_NEG_POS_RATIO * n_pos, float(n_p))
    hi0 = jnp.max(conf_neg) + 1.0

    def body(_, carry):
        lo, hi = carry
        t = 0.5 * (lo + hi)
        c = jnp.sum(jnp.where(conf_neg > t, 1.0, 0.0))
        geq = c >= k
        return jnp.where(geq, t, lo), jnp.where(geq, hi, t)

    lo, hi = lax.fori_loop(0, 40, body, (jnp.float32(-1.0), hi0))
    c_hi = jnp.sum(jnp.where(conf_neg > hi, 1.0, 0.0))
    s_hi = jnp.sum(jnp.where(conf_neg > hi, conf_neg, 0.0))
    v = jnp.maximum(lo, 0.0)
    hard = s_hi + jnp.maximum(k - c_hi, 0.0) * v
    hard = jnp.where(k > 0.5, hard, 0.0)

    i = pl.program_id(0)
    out_ref[i, 0] = n_pos
    out_ref[i, 1] = loc_sum
    out_ref[i, 2] = conf_pos_sum
    out_ref[i, 3] = hard


def kernel(predicted_locations, predicted_scores, boxes, labels, priors_cxcy):
    b, n_p, _ = predicted_locations.shape
    nobj = boxes.shape[1]
    plt = jnp.swapaxes(predicted_locations, 1, 2)
    labf = labels.astype(jnp.float32).reshape(b, nobj, 1)
    pcxcy_t = priors_cxcy.T
    pxy_t = jnp.concatenate(
        [pcxcy_t[:2] - pcxcy_t[2:] * 0.5, pcxcy_t[:2] + pcxcy_t[2:] * 0.5],
        axis=0)

    res = pl.pallas_call(
        _mbl_kernel,
        grid=(b,),
        in_specs=[
            pl.BlockSpec((1, 4, n_p), lambda i: (i, 0, 0)),
            pl.BlockSpec((1, n_p, predicted_scores.shape[2]),
                         lambda i: (i, 0, 0)),
            pl.BlockSpec((1, nobj, 4), lambda i: (i, 0, 0)),
            pl.BlockSpec((1, nobj, 1), lambda i: (i, 0, 0)),
            pl.BlockSpec((4, n_p), lambda i: (0, 0)),
            pl.BlockSpec((4, n_p), lambda i: (0, 0)),
        ],
        out_specs=pl.BlockSpec((b, 4), lambda i: (0, 0),
                               memory_space=pltpu.SMEM),
        out_shape=jax.ShapeDtypeStruct((b, 4), jnp.float32),
        compiler_params=pltpu.CompilerParams(
            vmem_limit_bytes=100 * 1024 * 1024),
    )(plt, predicted_scores, boxes, labf, pxy_t, pcxcy_t)

    n = jnp.sum(res[:, 0])
    loc_sum = jnp.sum(res[:, 1])
    conf_pos = jnp.sum(res[:, 2])
    hard = jnp.sum(res[:, 3])
    return (hard + conf_pos) / n + _ALPHA * loc_sum / (n * 4.0)


# speedup vs baseline: 10.7513x; 3.1373x over previous
import jax
import jax.numpy as jnp
from jax import lax
from jax.experimental import pallas as pl
from jax.experimental.pallas import tpu as pltpu

_THRESHOLD = 0.5
_NEG_POS_RATIO = 3.0
_ALPHA = 1.0


def _mbl_kernel(plt_ref, ps_ref, boxes_ref, labels_ref, pxy_ref, pcxcy_ref,
                out_ref):
    f32 = jnp.float32
    bx = boxes_ref[0]
    nobj = bx.shape[0]
    n_p = pxy_ref.shape[1]
    b1 = bx[:, 0:1]
    b2 = bx[:, 1:2]
    b3 = bx[:, 2:3]
    b4 = bx[:, 3:4]
    px1 = pxy_ref[0:1, :]
    py1 = pxy_ref[1:2, :]
    px2 = pxy_ref[2:3, :]
    py2 = pxy_ref[3:4, :]

    ix = jnp.maximum(jnp.minimum(b3, px2) - jnp.maximum(b1, px1), 0.0)
    iy = jnp.maximum(jnp.minimum(b4, py2) - jnp.maximum(b2, py1), 0.0)
    inter = ix * iy
    a1 = (b3 - b1) * (b4 - b2)
    a2 = (px2 - px1) * (py2 - py1)
    ov = inter / (a1 + a2 - inter)

    o_iota = lax.broadcasted_iota(jnp.int32, (nobj, 1), 0).astype(f32)
    p_iota = lax.broadcasted_iota(jnp.int32, (1, n_p), 1).astype(f32)

    ov_max0 = jnp.max(ov, axis=0, keepdims=True)
    ofe = jnp.min(jnp.where(ov == ov_max0, o_iota, float(nobj)),
                  axis=0, keepdims=True)
    ov_max1 = jnp.max(ov, axis=1, keepdims=True)
    pfeo = jnp.min(jnp.where(ov == ov_max1, p_iota, float(n_p) + 1.0),
                   axis=1, keepdims=True)

    eq_sc = p_iota == pfeo
    new_o = jnp.max(jnp.where(eq_sc, o_iota, -1.0), axis=0, keepdims=True)
    has = new_o >= 0.0
    ofe = jnp.where(has, new_o, ofe)
    ov_fe = jnp.where(has, 1.0, ov_max0)

    eq2 = ofe == o_iota

    def gather(v):
        return jnp.sum(jnp.where(eq2, v, 0.0), axis=0, keepdims=True)

    lab = gather(labels_ref[0])
    gcx = gather((b1 + b3) * 0.5)
    gcy = gather((b2 + b4) * 0.5)
    gw = gather(b3 - b1)
    gh = gather(b4 - b2)

    lab = jnp.where(ov_fe < _THRESHOLD, 0.0, lab)
    pos = jnp.where(lab > 0.5, 1.0, 0.0)
    n_pos = jnp.sum(pos)

    pcx = pcxcy_ref[0:1, :]
    pcy = pcxcy_ref[1:2, :]
    pw = pcxcy_ref[2:3, :]
    ph = pcxcy_ref[3:4, :]
    g1 = (gcx - pcx) / (pw * 0.1)
    g2 = (gcy - pcy) / (ph * 0.1)
    g3 = jnp.log(gw / pw) * 5.0
    g4 = jnp.log(gh / ph) * 5.0

    pl_t = plt_ref[0]
    loc = (jnp.abs(pl_t[0:1, :] - g1) + jnp.abs(pl_t[1:2, :] - g2) +
           jnp.abs(pl_t[2:3, :] - g3) + jnp.abs(pl_t[3:4, :] - g4))
    loc_sum = jnp.sum(loc * pos)

    lab_col = lab.reshape(n_p, 1)
    ps = ps_ref[0]
    m = jnp.max(ps, axis=1, keepdims=True)
    logz = m + jnp.log(jnp.sum(jnp.exp(ps - m), axis=1, keepdims=True))
    c_iota = lax.broadcasted_iota(jnp.int32, (1, ps.shape[1]), 1).astype(f32)
    gt_logit = jnp.sum(jnp.where(lab_col == c_iota, ps, 0.0),
                       axis=1, keepdims=True)
    conf = (logz - gt_logit).reshape(1, n_p)
    conf_pos_sum = jnp.sum(conf * pos)
    conf_neg = jnp.where(pos > 0.0, 0.0, conf)

    k = jnp.minimum(_NEG_POS_RATIO * n_pos, float(n_p))
    hi0 = jnp.max(conf_neg) + 1.0

    def body(_, carry):
        lo, hi = carry
        t = 0.5 * (lo + hi)
        c = jnp.sum(jnp.where(conf_neg > t, 1.0, 0.0))
        geq = c >= k
        return jnp.where(geq, t, lo), jnp.where(geq, hi, t)

    lo, hi = lax.fori_loop(0, 40, body, (jnp.float32(-1.0), hi0))
    c_hi = jnp.sum(jnp.where(conf_neg > hi, 1.0, 0.0))
    s_hi = jnp.sum(jnp.where(conf_neg > hi, conf_neg, 0.0))
    v = jnp.maximum(lo, 0.0)
    hard = s_hi + jnp.maximum(k - c_hi, 0.0) * v
    hard = jnp.where(k > 0.5, hard, 0.0)

    li = lax.broadcasted_iota(jnp.int32, (8, 128), 1)
    out_ref[0] = jnp.where(
        li == 0, n_pos,
        jnp.where(li == 1, loc_sum,
                  jnp.where(li == 2, conf_pos_sum,
                            jnp.where(li == 3, hard, 0.0))))


def kernel(predicted_locations, predicted_scores, boxes, labels, priors_cxcy):
    b, n_p, _ = predicted_locations.shape
    nobj = boxes.shape[1]
    plt = jnp.swapaxes(predicted_locations, 1, 2)
    labf = labels.astype(jnp.float32).reshape(b, nobj, 1)
    pcxcy_t = priors_cxcy.T
    pxy_t = jnp.concatenate(
        [pcxcy_t[:2] - pcxcy_t[2:] * 0.5, pcxcy_t[:2] + pcxcy_t[2:] * 0.5],
        axis=0)

    res = pl.pallas_call(
        _mbl_kernel,
        grid=(b,),
        in_specs=[
            pl.BlockSpec((1, 4, n_p), lambda i: (i, 0, 0)),
            pl.BlockSpec((1, n_p, predicted_scores.shape[2]),
                         lambda i: (i, 0, 0)),
            pl.BlockSpec((1, nobj, 4), lambda i: (i, 0, 0)),
            pl.BlockSpec((1, nobj, 1), lambda i: (i, 0, 0)),
            pl.BlockSpec((4, n_p), lambda i: (0, 0)),
            pl.BlockSpec((4, n_p), lambda i: (0, 0)),
        ],
        out_specs=pl.BlockSpec((1, 8, 128), lambda i: (i, 0, 0)),
        out_shape=jax.ShapeDtypeStruct((b, 8, 128), jnp.float32),
        compiler_params=pltpu.CompilerParams(
            vmem_limit_bytes=100 * 1024 * 1024,
            dimension_semantics=("parallel",)),
    )(plt, predicted_scores, boxes, labf, pxy_t, pcxcy_t)

    n = jnp.sum(res[:, 0, 0])
    loc_sum = jnp.sum(res[:, 0, 1])
    conf_pos = jnp.sum(res[:, 0, 2])
    hard = jnp.sum(res[:, 0, 3])
    return (hard + conf_pos) / n + _ALPHA * loc_sum / (n * 4.0)


# speedup vs baseline: 12.6463x; 1.1763x over previous
import jax
import jax.numpy as jnp
from jax import lax
from jax.experimental import pallas as pl
from jax.experimental.pallas import tpu as pltpu

_THRESHOLD = 0.5
_NEG_POS_RATIO = 3.0
_ALPHA = 1.0


def _mbl_kernel(plt_ref, ps_ref, boxes_ref, labels_ref, pxy_ref, pcxcy_ref,
                out_ref):
    f32 = jnp.float32
    bx = boxes_ref[0]
    nobj = bx.shape[0]
    n_p = pxy_ref.shape[1]
    b1 = bx[:, 0:1]
    b2 = bx[:, 1:2]
    b3 = bx[:, 2:3]
    b4 = bx[:, 3:4]
    px1 = pxy_ref[0:1, :]
    py1 = pxy_ref[1:2, :]
    px2 = pxy_ref[2:3, :]
    py2 = pxy_ref[3:4, :]

    ix = jnp.maximum(jnp.minimum(b3, px2) - jnp.maximum(b1, px1), 0.0)
    iy = jnp.maximum(jnp.minimum(b4, py2) - jnp.maximum(b2, py1), 0.0)
    inter = ix * iy
    a1 = (b3 - b1) * (b4 - b2)
    a2 = (px2 - px1) * (py2 - py1)
    ov = inter / (a1 + a2 - inter)

    o_iota = lax.broadcasted_iota(jnp.int32, (nobj, 1), 0).astype(f32)
    p_iota = lax.broadcasted_iota(jnp.int32, (1, n_p), 1).astype(f32)

    ov_max0 = jnp.max(ov, axis=0, keepdims=True)
    ofe = jnp.min(jnp.where(ov == ov_max0, o_iota, float(nobj)),
                  axis=0, keepdims=True)
    ov_max1 = jnp.max(ov, axis=1, keepdims=True)
    pfeo = jnp.min(jnp.where(ov == ov_max1, p_iota, float(n_p) + 1.0),
                   axis=1, keepdims=True)

    eq_sc = p_iota == pfeo
    new_o = jnp.max(jnp.where(eq_sc, o_iota, -1.0), axis=0, keepdims=True)
    has = new_o >= 0.0
    ofe = jnp.where(has, new_o, ofe)
    ov_fe = jnp.where(has, 1.0, ov_max0)

    eq2 = ofe == o_iota

    def gather(v):
        return jnp.sum(jnp.where(eq2, v, 0.0), axis=0, keepdims=True)

    lab = gather(labels_ref[0])
    gcx = gather((b1 + b3) * 0.5)
    gcy = gather((b2 + b4) * 0.5)
    gw = gather(b3 - b1)
    gh = gather(b4 - b2)

    lab = jnp.where(ov_fe < _THRESHOLD, 0.0, lab)
    pos = jnp.where(lab > 0.5, 1.0, 0.0)
    n_pos = jnp.sum(pos)

    pcx = pcxcy_ref[0:1, :]
    pcy = pcxcy_ref[1:2, :]
    pw = pcxcy_ref[2:3, :]
    ph = pcxcy_ref[3:4, :]
    g1 = (gcx - pcx) / (pw * 0.1)
    g2 = (gcy - pcy) / (ph * 0.1)
    g3 = jnp.log(gw / pw) * 5.0
    g4 = jnp.log(gh / ph) * 5.0

    pl_t = plt_ref[0]
    loc = (jnp.abs(pl_t[0:1, :] - g1) + jnp.abs(pl_t[1:2, :] - g2) +
           jnp.abs(pl_t[2:3, :] - g3) + jnp.abs(pl_t[3:4, :] - g4))
    loc_sum = jnp.sum(loc * pos)

    lab_col = lab.reshape(n_p, 1)
    ps = ps_ref[0]
    m = jnp.max(ps, axis=1, keepdims=True)
    logz = m + jnp.log(jnp.sum(jnp.exp(ps - m), axis=1, keepdims=True))
    c_iota = lax.broadcasted_iota(jnp.int32, (1, ps.shape[1]), 1).astype(f32)
    gt_logit = jnp.sum(jnp.where(lab_col == c_iota, ps, 0.0),
                       axis=1, keepdims=True)
    conf = (logz - gt_logit).reshape(1, n_p)
    conf_pos_sum = jnp.sum(conf * pos)
    conf_neg = jnp.where(pos > 0.0, 0.0, conf)

    k = jnp.minimum(_NEG_POS_RATIO * n_pos, float(n_p))
    hi0 = jnp.max(conf_neg) + 1.0

    def body(_, carry):
        lo, hi = carry
        d = (hi - lo) * 0.25
        t1 = lo + d
        t2 = lo + 2.0 * d
        t3 = lo + 3.0 * d
        c1 = jnp.sum(jnp.where(conf_neg > t1, 1.0, 0.0))
        c2 = jnp.sum(jnp.where(conf_neg > t2, 1.0, 0.0))
        c3 = jnp.sum(jnp.where(conf_neg > t3, 1.0, 0.0))
        new_lo = jnp.where(
            c1 >= k, jnp.where(c2 >= k, jnp.where(c3 >= k, t3, t2), t1), lo)
        new_hi = jnp.where(
            c1 < k, t1, jnp.where(c2 < k, t2, jnp.where(c3 < k, t3, hi)))
        return new_lo, new_hi

    lo, hi = lax.fori_loop(0, 14, body, (jnp.float32(-1.0), hi0))
    c_hi = jnp.sum(jnp.where(conf_neg > hi, 1.0, 0.0))
    s_hi = jnp.sum(jnp.where(conf_neg > hi, conf_neg, 0.0))
    v = jnp.maximum(lo, 0.0)
    hard = s_hi + jnp.maximum(k - c_hi, 0.0) * v
    hard = jnp.where(k > 0.5, hard, 0.0)

    li = lax.broadcasted_iota(jnp.int32, (8, 128), 1)
    out_ref[0] = jnp.where(
        li == 0, n_pos,
        jnp.where(li == 1, loc_sum,
                  jnp.where(li == 2, conf_pos_sum,
                            jnp.where(li == 3, hard, 0.0))))


def kernel(predicted_locations, predicted_scores, boxes, labels, priors_cxcy):
    b, n_p, _ = predicted_locations.shape
    nobj = boxes.shape[1]
    plt = jnp.swapaxes(predicted_locations, 1, 2)
    labf = labels.astype(jnp.float32).reshape(b, nobj, 1)
    pcxcy_t = priors_cxcy.T
    pxy_t = jnp.concatenate(
        [pcxcy_t[:2] - pcxcy_t[2:] * 0.5, pcxcy_t[:2] + pcxcy_t[2:] * 0.5],
        axis=0)

    res = pl.pallas_call(
        _mbl_kernel,
        grid=(b,),
        in_specs=[
            pl.BlockSpec((1, 4, n_p), lambda i: (i, 0, 0)),
            pl.BlockSpec((1, n_p, predicted_scores.shape[2]),
                         lambda i: (i, 0, 0)),
            pl.BlockSpec((1, nobj, 4), lambda i: (i, 0, 0)),
            pl.BlockSpec((1, nobj, 1), lambda i: (i, 0, 0)),
            pl.BlockSpec((4, n_p), lambda i: (0, 0)),
            pl.BlockSpec((4, n_p), lambda i: (0, 0)),
        ],
        out_specs=pl.BlockSpec((1, 8, 128), lambda i: (i, 0, 0)),
        out_shape=jax.ShapeDtypeStruct((b, 8, 128), jnp.float32),
        compiler_params=pltpu.CompilerParams(
            vmem_limit_bytes=100 * 1024 * 1024,
            dimension_semantics=("parallel",)),
    )(plt, predicted_scores, boxes, labf, pxy_t, pcxcy_t)

    n = jnp.sum(res[:, 0, 0])
    loc_sum = jnp.sum(res[:, 0, 1])
    conf_pos = jnp.sum(res[:, 0, 2])
    hard = jnp.sum(res[:, 0, 3])
    return (hard + conf_pos) / n + _ALPHA * loc_sum / (n * 4.0)
